# Initial kernel scaffold; baseline (speedup 1.0000x reference)
import jax
import jax.numpy as jnp
from jax.experimental import pallas as pl


def kernel(vec_seq, base_attn_mask, Wq, bq, Wk, bk):
    raise NotImplementedError("write your pallas kernel here")


# speedup vs baseline: 1.2658x; 1.2658x over previous
import math

import jax
import jax.numpy as jnp
import numpy as np
from jax.experimental import pallas as pl

RSCALE = np.float32(0.141421363)
BQ = 128


def _proj_kernel(v_ref, wq_ref, bq_ref, wk_ref, bk_ref, q_ref, k_ref):
    v = v_ref[0]
    accq = jax.lax.dot_general(wq_ref[...], v, (((1,), (1,)), ((), ())),
                               preferred_element_type=jnp.float32)
    q_ref[0] = (accq + bq_ref[...]).astype(jnp.bfloat16)
    acck = jax.lax.dot_general(wk_ref[...], v, (((1,), (1,)), ((), ())),
                               preferred_element_type=jnp.float32)
    k_ref[0] = (acck + bk_ref[...]).astype(jnp.bfloat16)


def _structured_rowsum(x, s):
    r = x.reshape(s // 8, 8, x.shape[-1])
    acc = jnp.sum(r, axis=0)
    a4 = acc[:4] + acc[4:]
    a2 = a4[:2] + a4[2:]
    return a2[0:1] + a2[1:2]


def _attn_kernel(k_ref, q_ref, ts_ref):
    kk = k_ref[0]
    qq = q_ref[0]
    s = kk.shape[-1]
    sT = jax.lax.dot_general(kk, qq, (((0,), (0,)), ((), ())),
                             preferred_element_type=jnp.float32)
    sT = sT * RSCALE
    m = jnp.max(sT, axis=0, keepdims=True)
    u = jnp.exp(sT - m)
    den = _structured_rowsum(u, s)
    p = u / den
    ts_ref[0, 0] = _structured_rowsum(p, s)


def _count_ge(xi, mid):
    return jnp.sum((xi >= mid).astype(jnp.int32))


def _select_kernel(ts_ref, mask_ref, v_ref, tsn_ref, ep_ref, ap_ref, anp_ref):
    ts = ts_ref[0]
    mask = mask_ref[0]
    s = ts.shape[-1]

    num_tokens = jnp.sum(mask)
    k = (np.float32(0.2) * num_tokens).astype(jnp.int32)

    tssum = jnp.sum(ts)
    tsn = ts / tssum

    xi = jax.lax.bitcast_convert_type(tsn, jnp.int32)

    def vstep(_, lh):
        lo, hi = lh
        mid = lo + (hi - lo + 1) // 2
        c = _count_ge(xi, mid)
        big = c >= k
        return (jnp.where(big, mid, lo), jnp.where(big, hi, mid - 1))

    lo, hi = jax.lax.fori_loop(0, 32, vstep,
                               (jnp.int32(0), jnp.int32(2**31 - 2)))
    vthr = lo

    gt = xi > vthr
    tie = xi == vthr
    n_gt = jnp.sum(gt.astype(jnp.int32))
    tn = k - n_gt

    iota = jax.lax.broadcasted_iota(jnp.int32, (1, s), 1)

    def jstep(_, lh):
        lo_, hi_ = lh
        mid = (lo_ + hi_) // 2
        f = jnp.sum((tie & (iota <= mid)).astype(jnp.int32))
        ok = f >= tn
        return (jnp.where(ok, lo_, mid), jnp.where(ok, mid, hi_))

    jlo, jhi = jax.lax.fori_loop(0, 13, jstep,
                                 (jnp.int32(-1), jnp.int32(s - 1)))
    expl = (gt | (tie & (iota <= jhi) & (tn > 0))).astype(jnp.float32)

    anti = 1.0 - expl
    se = jnp.sum(expl * tsn)
    sa = jnp.sum(anti * tsn)
    w_exp = expl * tsn / se
    w_anti = anti * tsn / sa

    v = v_ref[0]
    w = jnp.concatenate([w_exp, tsn, w_anti], axis=0)
    pools = jax.lax.dot_general(w, v, (((1,), (0,)), ((), ())),
                                preferred_element_type=jnp.float32)
    tsn_ref[0] = tsn
    ep_ref[0] = pools[0:1]
    ap_ref[0] = pools[1:2]
    anp_ref[0] = pools[2:3]


def kernel(vec_seq, base_attn_mask, Wq, bq, Wk, bk):
    B, S, H = vec_seq.shape
    D = Wq.shape[0]

    qbT, kbT = pl.pallas_call(
        _proj_kernel,
        grid=(B,),
        in_specs=[
            pl.BlockSpec((1, S, H), lambda b: (b, 0, 0)),
            pl.BlockSpec((D, H), lambda b: (0, 0)),
            pl.BlockSpec((D, 1), lambda b: (0, 0)),
            pl.BlockSpec((D, H), lambda b: (0, 0)),
            pl.BlockSpec((D, 1), lambda b: (0, 0)),
        ],
        out_specs=[pl.BlockSpec((1, D, S), lambda b: (b, 0, 0))] * 2,
        out_shape=[jax.ShapeDtypeStruct((B, D, S), jnp.bfloat16)] * 2,
    )(vec_seq, Wq, bq[:, None], Wk, bk[:, None])

    ts = pl.pallas_call(
        _attn_kernel,
        grid=(B, S // BQ),
        in_specs=[
            pl.BlockSpec((1, D, S), lambda b, i: (b, 0, 0)),
            pl.BlockSpec((1, D, BQ), lambda b, i: (b, 0, i)),
        ],
        out_specs=pl.BlockSpec((1, 1, 1, BQ), lambda b, i: (b, i, 0, 0)),
        out_shape=jax.ShapeDtypeStruct((B, S // BQ, 1, BQ), jnp.float32),
    )(kbT, qbT).reshape(B, 1, S)

    tsn, ep, ap, anp = pl.pallas_call(
        _select_kernel,
        grid=(B,),
        in_specs=[
            pl.BlockSpec((1, 1, S), lambda b: (b, 0, 0)),
            pl.BlockSpec((1, 1, S), lambda b: (b, 0, 0)),
            pl.BlockSpec((1, S, H), lambda b: (b, 0, 0)),
        ],
        out_specs=[
            pl.BlockSpec((1, 1, S), lambda b: (b, 0, 0)),
            pl.BlockSpec((1, 1, H), lambda b: (b, 0, 0)),
            pl.BlockSpec((1, 1, H), lambda b: (b, 0, 0)),
            pl.BlockSpec((1, 1, H), lambda b: (b, 0, 0)),
        ],
        out_shape=[
            jax.ShapeDtypeStruct((B, 1, S), jnp.float32),
            jax.ShapeDtypeStruct((B, 1, H), jnp.float32),
            jax.ShapeDtypeStruct((B, 1, H), jnp.float32),
            jax.ShapeDtypeStruct((B, 1, H), jnp.float32),
        ],
    )(ts, base_attn_mask.reshape(B, 1, S), vec_seq)

    return (tsn.reshape(B, S), ep.reshape(B, H),
            ap.reshape(B, H), anp.reshape(B, H))


# speedup vs baseline: 1.2668x; 1.0008x over previous
import math

import jax
import jax.numpy as jnp
import numpy as np
from jax.experimental import pallas as pl

RSCALE = np.float32(0.141421363)
BQ = 128


def _proj_kernel(v_ref, wq_ref, bq_ref, wk_ref, bk_ref, q_ref, k_ref):
    v = v_ref[0]
    accq = jax.lax.dot_general(wq_ref[...], v, (((1,), (1,)), ((), ())),
                               preferred_element_type=jnp.float32)
    q_ref[0] = (accq + bq_ref[...]).astype(jnp.bfloat16)
    acck = jax.lax.dot_general(wk_ref[...], v, (((1,), (1,)), ((), ())),
                               preferred_element_type=jnp.float32)
    k_ref[0] = (acck + bk_ref[...]).astype(jnp.bfloat16)


def _structured_rowsum(x, s):
    r = x.reshape(s // 8, 8, x.shape[-1])
    acc = jnp.sum(r, axis=0)
    a4 = acc[:4] + acc[4:]
    a2 = a4[:2] + a4[2:]
    return a2[0:1] + a2[1:2]


def _attn_kernel(k_ref, q_ref, ts_ref):
    kk = k_ref[0]
    qq = q_ref[0]
    s = kk.shape[-1]
    sT = jax.lax.dot_general(kk, qq, (((0,), (0,)), ((), ())),
                             preferred_element_type=jnp.float32)
    sT = sT * RSCALE
    m = jnp.max(sT, axis=0, keepdims=True)
    u = jnp.exp(sT - m)
    den = _structured_rowsum(u, s)
    p = u * (np.float32(1.0) / den)
    ts_ref[0, 0] = _structured_rowsum(p, s)


def _count_ge(xi, mid):
    return jnp.sum((xi >= mid).astype(jnp.int32))


def _select_kernel(ts_ref, mask_ref, v_ref, tsn_ref, ep_ref, ap_ref, anp_ref):
    ts = ts_ref[0]
    mask = mask_ref[0]
    s = ts.shape[-1]

    num_tokens = jnp.sum(mask)
    k = (np.float32(0.2) * num_tokens).astype(jnp.int32)

    tssum = jnp.sum(ts)
    tsn = ts / tssum

    xi = jax.lax.bitcast_convert_type(tsn, jnp.int32)

    def vstep(_, lh):
        lo, hi = lh
        mid = lo + (hi - lo + 1) // 2
        c = _count_ge(xi, mid)
        big = c >= k
        return (jnp.where(big, mid, lo), jnp.where(big, hi, mid - 1))

    lo, hi = jax.lax.fori_loop(0, 32, vstep,
                               (jnp.int32(0), jnp.int32(2**31 - 2)))
    vthr = lo

    gt = xi > vthr
    tie = xi == vthr
    n_gt = jnp.sum(gt.astype(jnp.int32))
    tn = k - n_gt

    iota = jax.lax.broadcasted_iota(jnp.int32, (1, s), 1)

    def jstep(_, lh):
        lo_, hi_ = lh
        mid = (lo_ + hi_) // 2
        f = jnp.sum((tie & (iota <= mid)).astype(jnp.int32))
        ok = f >= tn
        return (jnp.where(ok, lo_, mid), jnp.where(ok, mid, hi_))

    jlo, jhi = jax.lax.fori_loop(0, 13, jstep,
                                 (jnp.int32(-1), jnp.int32(s - 1)))
    expl = (gt | (tie & (iota <= jhi) & (tn > 0))).astype(jnp.float32)

    anti = 1.0 - expl
    se = jnp.sum(expl * tsn)
    sa = jnp.sum(anti * tsn)
    w_exp = expl * tsn / se
    w_anti = anti * tsn / sa

    v = v_ref[0]
    w = jnp.concatenate([w_exp, tsn, w_anti], axis=0)
    pools = jax.lax.dot_general(w, v, (((1,), (0,)), ((), ())),
                                preferred_element_type=jnp.float32)
    tsn_ref[0] = tsn
    ep_ref[0] = pools[0:1]
    ap_ref[0] = pools[1:2]
    anp_ref[0] = pools[2:3]


def kernel(vec_seq, base_attn_mask, Wq, bq, Wk, bk):
    B, S, H = vec_seq.shape
    D = Wq.shape[0]

    qbT, kbT = pl.pallas_call(
        _proj_kernel,
        grid=(B,),
        in_specs=[
            pl.BlockSpec((1, S, H), lambda b: (b, 0, 0)),
            pl.BlockSpec((D, H), lambda b: (0, 0)),
            pl.BlockSpec((D, 1), lambda b: (0, 0)),
            pl.BlockSpec((D, H), lambda b: (0, 0)),
            pl.BlockSpec((D, 1), lambda b: (0, 0)),
        ],
        out_specs=[pl.BlockSpec((1, D, S), lambda b: (b, 0, 0))] * 2,
        out_shape=[jax.ShapeDtypeStruct((B, D, S), jnp.bfloat16)] * 2,
    )(vec_seq, Wq, bq[:, None], Wk, bk[:, None])

    ts = pl.pallas_call(
        _attn_kernel,
        grid=(B, S // BQ),
        in_specs=[
            pl.BlockSpec((1, D, S), lambda b, i: (b, 0, 0)),
            pl.BlockSpec((1, D, BQ), lambda b, i: (b, 0, i)),
        ],
        out_specs=pl.BlockSpec((1, 1, 1, BQ), lambda b, i: (b, i, 0, 0)),
        out_shape=jax.ShapeDtypeStruct((B, S // BQ, 1, BQ), jnp.float32),
    )(kbT, qbT).reshape(B, 1, S)

    tsn, ep, ap, anp = pl.pallas_call(
        _select_kernel,
        grid=(B,),
        in_specs=[
            pl.BlockSpec((1, 1, S), lambda b: (b, 0, 0)),
            pl.BlockSpec((1, 1, S), lambda b: (b, 0, 0)),
            pl.BlockSpec((1, S, H), lambda b: (b, 0, 0)),
        ],
        out_specs=[
            pl.BlockSpec((1, 1, S), lambda b: (b, 0, 0)),
            pl.BlockSpec((1, 1, H), lambda b: (b, 0, 0)),
            pl.BlockSpec((1, 1, H), lambda b: (b, 0, 0)),
            pl.BlockSpec((1, 1, H), lambda b: (b, 0, 0)),
        ],
        out_shape=[
            jax.ShapeDtypeStruct((B, 1, S), jnp.float32),
            jax.ShapeDtypeStruct((B, 1, H), jnp.float32),
            jax.ShapeDtypeStruct((B, 1, H), jnp.float32),
            jax.ShapeDtypeStruct((B, 1, H), jnp.float32),
        ],
    )(ts, base_attn_mask.reshape(B, 1, S), vec_seq)

    return (tsn.reshape(B, S), ep.reshape(B, H),
            ap.reshape(B, H), anp.reshape(B, H))


# speedup vs baseline: 2.5068x; 1.9788x over previous
import math

import jax
import jax.numpy as jnp
import numpy as np
from jax.experimental import pallas as pl

RSCALE = np.float32(0.141421363)


def _structured_rowsum(x, s):
    r = x.reshape(s // 8, 8, x.shape[-1])
    acc = jnp.sum(r, axis=0)
    a4 = acc[:4] + acc[4:]
    a2 = a4[:2] + a4[2:]
    return a2[0:1] + a2[1:2]


def _fused_kernel(v_ref, mask_ref, wq_ref, bq_ref, wk_ref, bk_ref,
                  tsn_ref, ep_ref, ap_ref, anp_ref):
    v = v_ref[0]
    s = v.shape[0]

    qT = (jax.lax.dot_general(wq_ref[...], v, (((1,), (1,)), ((), ())),
                              preferred_element_type=jnp.float32)
          + bq_ref[...]).astype(jnp.bfloat16)
    kT = (jax.lax.dot_general(wk_ref[...], v, (((1,), (1,)), ((), ())),
                              preferred_element_type=jnp.float32)
          + bk_ref[...]).astype(jnp.bfloat16)

    sT = jax.lax.dot_general(kT, qT, (((0,), (0,)), ((), ())),
                             preferred_element_type=jnp.float32)
    sT = sT * RSCALE
    m = jnp.max(sT, axis=0, keepdims=True)
    u = jnp.exp(sT - m)
    den = _structured_rowsum(u, s)
    p = u * (np.float32(1.0) / den)
    ts = _structured_rowsum(p, s)

    mask = mask_ref[0]
    num_tokens = jnp.sum(mask)
    k = (np.float32(0.2) * num_tokens).astype(jnp.int32)

    tssum = jnp.sum(ts)
    tsn = ts / tssum

    xi = jax.lax.bitcast_convert_type(tsn, jnp.int32)

    def vstep(_, lh):
        lo, hi = lh
        mid = lo + (hi - lo + 1) // 2
        c = jnp.sum((xi >= mid).astype(jnp.int32))
        big = c >= k
        return (jnp.where(big, mid, lo), jnp.where(big, hi, mid - 1))

    vthr, _ = jax.lax.fori_loop(0, 32, vstep,
                                (jnp.int32(0), jnp.int32(2**31 - 2)))

    gt = xi > vthr
    tie = xi == vthr
    n_gt = jnp.sum(gt.astype(jnp.int32))
    tn = k - n_gt

    iota = jax.lax.broadcasted_iota(jnp.int32, (1, s), 1)

    def jstep(_, lh):
        lo_, hi_ = lh
        mid = (lo_ + hi_) // 2
        f = jnp.sum((tie & (iota <= mid)).astype(jnp.int32))
        ok = f >= tn
        return (jnp.where(ok, lo_, mid), jnp.where(ok, mid, hi_))

    _, jhi = jax.lax.fori_loop(0, 13, jstep,
                               (jnp.int32(-1), jnp.int32(s - 1)))
    expl = (gt | (tie & (iota <= jhi) & (tn > 0))).astype(jnp.float32)

    anti = 1.0 - expl
    se = jnp.sum(expl * tsn)
    sa = jnp.sum(anti * tsn)
    w = jnp.concatenate([expl * tsn / se, tsn, anti * tsn / sa], axis=0)
    pools = jax.lax.dot_general(w, v, (((1,), (0,)), ((), ())),
                                preferred_element_type=jnp.float32)
    tsn_ref[0] = tsn
    ep_ref[0] = pools[0:1]
    ap_ref[0] = pools[1:2]
    anp_ref[0] = pools[2:3]


def kernel(vec_seq, base_attn_mask, Wq, bq, Wk, bk):
    B, S, H = vec_seq.shape
    D = Wq.shape[0]

    tsn, ep, ap, anp = pl.pallas_call(
        _fused_kernel,
        grid=(B,),
        in_specs=[
            pl.BlockSpec((1, S, H), lambda b: (b, 0, 0)),
            pl.BlockSpec((1, 1, S), lambda b: (b, 0, 0)),
            pl.BlockSpec((D, H), lambda b: (0, 0)),
            pl.BlockSpec((D, 1), lambda b: (0, 0)),
            pl.BlockSpec((D, H), lambda b: (0, 0)),
            pl.BlockSpec((D, 1), lambda b: (0, 0)),
        ],
        out_specs=[
            pl.BlockSpec((1, 1, S), lambda b: (b, 0, 0)),
            pl.BlockSpec((1, 1, H), lambda b: (b, 0, 0)),
            pl.BlockSpec((1, 1, H), lambda b: (b, 0, 0)),
            pl.BlockSpec((1, 1, H), lambda b: (b, 0, 0)),
        ],
        out_shape=[
            jax.ShapeDtypeStruct((B, 1, S), jnp.float32),
            jax.ShapeDtypeStruct((B, 1, H), jnp.float32),
            jax.ShapeDtypeStruct((B, 1, H), jnp.float32),
            jax.ShapeDtypeStruct((B, 1, H), jnp.float32),
        ],
    )(vec_seq, base_attn_mask.reshape(B, 1, S), Wq, bq[:, None], Wk, bk[:, None])

    return (tsn.reshape(B, S), ep.reshape(B, H),
            ap.reshape(B, H), anp.reshape(B, H))
